# baseline (speedup 1.0000x reference)
import functools

import jax
import jax.numpy as jnp
from jax import lax
from jax.experimental import pallas as pl
from jax.experimental.pallas import tpu as pltpu
from jax.experimental.pallas import tpu_sc as plsc

N = 10000
E = 320000
NC = 2
NS = 16
NW = NC * NS
EPW = E // NW
LANES = 16
CHUNKS = EPW // LANES

_SC_MESH = dict(core_axis_name="c", subcore_axis_name="s",
                num_cores=NC, num_subcores=NS)
_SC_PARAMS = pltpu.CompilerParams(needs_layout_passes=False)


def _wid():
    return lax.axis_index("s") * NC + lax.axis_index("c")


def _deg_body(dst_hbm, out_hbm, dst_v, acc_v):
    w = _wid()
    pltpu.sync_copy(dst_hbm.at[w], dst_v)
    one = jnp.ones((LANES,), jnp.float32)
    zero = jnp.zeros((LANES,), jnp.float32)

    @plsc.parallel_loop(0, N // LANES, unroll=8)
    def _(i):
        acc_v[pl.ds(i * LANES, LANES)] = zero

    @plsc.parallel_loop(0, CHUNKS, unroll=8)
    def _(i):
        d = dst_v[pl.ds(i * LANES, LANES)]
        plsc.addupdate_scatter(acc_v, [d], one)

    pltpu.sync_copy(acc_v, out_hbm.at[w])


_deg_kernel = functools.partial(
    pl.kernel,
    out_type=jax.ShapeDtypeStruct((NW, N), jnp.float32),
    mesh=plsc.VectorSubcoreMesh(**_SC_MESH),
    compiler_params=_SC_PARAMS,
    scratch_types=[
        pltpu.VMEM((EPW,), jnp.int32),
        pltpu.VMEM((N,), jnp.float32),
    ],
)(_deg_body)


def _agg_body(k_planes, *refs):
    hwt_hbm, src_hbm, dst_hbm = refs[:3]
    outs = refs[3:3 + k_planes]
    src_v, dst_v = refs[3 + k_planes:5 + k_planes]
    hw_vs = refs[5 + k_planes:5 + 2 * k_planes]
    acc_vs = refs[5 + 2 * k_planes:5 + 3 * k_planes]
    w = _wid()
    pltpu.sync_copy(src_hbm.at[w], src_v)
    pltpu.sync_copy(dst_hbm.at[w], dst_v)
    for k in range(k_planes):
        pltpu.sync_copy(hwt_hbm.at[k], hw_vs[k])

    zero = jnp.zeros((LANES,), jnp.float32)

    @plsc.parallel_loop(0, N // LANES, unroll=8)
    def _(i):
        for k in range(k_planes):
            acc_vs[k][pl.ds(i * LANES, LANES)] = zero

    @plsc.parallel_loop(0, CHUNKS, unroll=8)
    def _(i):
        sl = pl.ds(i * LANES, LANES)
        vs = src_v[sl]
        vd = dst_v[sl]
        for k in range(k_planes):
            g = plsc.load_gather(hw_vs[k], [vs])
            plsc.addupdate_scatter(acc_vs[k], [vd], g)

    for k in range(k_planes):
        pltpu.sync_copy(acc_vs[k], outs[k].at[w])


def _make_agg_kernel(k_planes):
    return functools.partial(
        pl.kernel,
        out_type=tuple(jax.ShapeDtypeStruct((NW, N), jnp.float32)
                       for _ in range(k_planes)),
        mesh=plsc.VectorSubcoreMesh(**_SC_MESH),
        compiler_params=_SC_PARAMS,
        scratch_types=(
            [pltpu.VMEM((EPW,), jnp.int32)] * 2
            + [pltpu.VMEM((N,), jnp.float32)] * (2 * k_planes)
        ),
    )(functools.partial(_agg_body, k_planes))


_agg4 = _make_agg_kernel(4)
_agg2 = _make_agg_kernel(2)


def _prep_body(dp_ref, x_ref, w1_ref, dinv_ref, hwt1_ref):
    deg = jnp.ones((N,), jnp.float32) + jnp.sum(dp_ref[...], axis=0)
    dinv = lax.rsqrt(deg)
    dinv_ref[...] = dinv
    hw1 = jnp.dot(x_ref[...], w1_ref[...], preferred_element_type=jnp.float32)
    hwt1_ref[...] = hw1 * dinv[:, None]


def _tc_prep(deg_partials, x, W1):
    return pl.pallas_call(
        _prep_body,
        out_shape=(
            jax.ShapeDtypeStruct((N,), jnp.float32),
            jax.ShapeDtypeStruct((N, 4), jnp.float32),
        ),
    )(deg_partials, x, W1)


def _dense_body(k_in, k_out, scale_out, refs):
    ps = refs[:k_in]
    hwt_ref, dinv_ref, b_ref, wT_ref, bo_ref = refs[k_in:k_in + 5]
    h_ref, hwtn_ref = refs[k_in + 5:]
    dinv = dinv_ref[...]
    hs = []
    for k in range(k_in):
        agg = jnp.sum(ps[k][...], axis=0) + hwt_ref[k]
        hs.append(jnp.tanh(dinv * agg + b_ref[0, k]))
        h_ref[k] = hs[k]
    for j in range(k_out):
        acc = bo_ref[0, j] + jnp.zeros((N,), jnp.float32)
        for k in range(k_in):
            acc = acc + wT_ref[j, k] * hs[k]
        hwtn_ref[j] = dinv * acc if scale_out else acc


def _tc_dense(k_in, k_out, scale_out, partials, hwt, dinv, b, WT, bo):
    smem = pl.BlockSpec(memory_space=pltpu.SMEM)
    body = lambda *refs: _dense_body(k_in, k_out, scale_out, refs)
    return pl.pallas_call(
        body,
        in_specs=[pl.BlockSpec() for _ in partials]
                 + [pl.BlockSpec(), pl.BlockSpec(), smem, smem, smem],
        out_shape=(
            jax.ShapeDtypeStruct((k_in, N), jnp.float32),
            jax.ShapeDtypeStruct((k_out, N), jnp.float32),
        ),
    )(*partials, hwt, dinv, b, WT, bo)


def kernel(x, edge_index, W1, b1, W2, b2, W3, b3, Wc, bc):
    src = edge_index[0].reshape(NW, EPW)
    dst = edge_index[1].reshape(NW, EPW)

    deg_partials = _deg_kernel(dst)
    dinv, hwt1_rows = _tc_prep(deg_partials, x, W1)
    hwt1 = hwt1_rows.T

    z4 = jnp.zeros((1, 4), jnp.float32)
    z2 = jnp.zeros((1, 2), jnp.float32)

    p1 = _agg4(hwt1, src, dst)
    h1, hwt2 = _tc_dense(4, 4, True, p1, hwt1, dinv,
                         b1.reshape(1, 4), W2.T, z4)

    p2 = _agg4(hwt2, src, dst)
    h2, hwt3 = _tc_dense(4, 2, True, p2, hwt2, dinv,
                         b2.reshape(1, 4), W3.T, z2)

    p3 = _agg2(hwt3, src, dst)
    h3, outp = _tc_dense(2, 10, False, p3, hwt3, dinv,
                         b3.reshape(1, 2), Wc.T, bc.reshape(1, 10))

    return (outp.T, h3.T)
